# baseline (speedup 1.0000x reference)
import functools

import jax
import jax.numpy as jnp
from jax import lax
from jax.experimental import pallas as pl
from jax.experimental.pallas import tpu as pltpu
from jax.experimental.pallas import tpu_sc as plsc


def _mlp_table_body(tab_ref, w1_ref, b1_ref, w2_ref, b2_ref, y_ref):
    h = jnp.dot(tab_ref[...], w1_ref[...], preferred_element_type=jnp.float32)
    h = jnp.maximum(h + b1_ref[...], 0.0)
    y_ref[...] = (
        jnp.dot(h, w2_ref[...], preferred_element_type=jnp.float32) + b2_ref[...]
    )


def _compute_vocab_outputs(embed_table, W1, b1, W2, b2):
    V = embed_table.shape[0]
    H = W1.shape[1]
    O = W2.shape[1]
    return pl.pallas_call(
        _mlp_table_body,
        out_shape=jax.ShapeDtypeStruct((V, O), jnp.float32),
    )(embed_table, W1, b1.reshape(1, H), W2, b2.reshape(1, O))


@functools.cache
def _make_gather(V, D, N):
    info = plsc.get_sparse_core_info()
    NC, NS = info.num_cores, info.num_subcores
    NW = NC * NS
    assert N % NW == 0
    n_per = N // NW
    NBUF = 4
    C = 80
    assert n_per % (NBUF * C) == 0
    rounds = n_per // (NBUF * C)

    mesh = plsc.VectorSubcoreMesh(core_axis_name="c", subcore_axis_name="s")

    @functools.partial(
        pl.kernel,
        out_type=jax.ShapeDtypeStruct((N, D), jnp.float32),
        mesh=mesh,
        scratch_types=[
            pltpu.VMEM((n_per,), jnp.int32),
        ]
        + [pltpu.VMEM((C, D), jnp.float32)] * NBUF
        + [pltpu.SemaphoreType.DMA] * (2 * NBUF),
    )
    def gather(y_hbm, idx_hbm, out_hbm, idx_v, *bufs_and_sems):
        rows = bufs_and_sems[:NBUF]
        gsem = bufs_and_sems[NBUF : 2 * NBUF]
        ssem = bufs_and_sems[2 * NBUF :]
        wid = lax.axis_index("s") * NC + lax.axis_index("c")
        base = wid * n_per
        pltpu.sync_copy(idx_hbm.at[pl.ds(base, n_per)], idx_v)

        def body(i, carry):
            for j in range(NBUF):
                g = i * NBUF + j

                @pl.when(i > 0)
                def _(j=j, g=g):
                    pltpu.make_async_copy(
                        rows[j], out_hbm.at[pl.ds(base + (g - NBUF) * C, C)], ssem[j]
                    ).wait()

                pltpu.async_copy(
                    y_hbm.at[idx_v.at[pl.ds(g * C, C)]], rows[j], gsem[j]
                )
            for j in range(NBUF):
                g = i * NBUF + j
                pltpu.make_async_copy(
                    y_hbm.at[idx_v.at[pl.ds(0, C)]], rows[j], gsem[j]
                ).wait()
                pltpu.async_copy(
                    rows[j], out_hbm.at[pl.ds(base + g * C, C)], ssem[j]
                )
            return carry

        lax.fori_loop(0, rounds, body, 0)

        for j in range(NBUF):
            g = (rounds - 1) * NBUF + j
            pltpu.make_async_copy(
                rows[j], out_hbm.at[pl.ds(base + g * C, C)], ssem[j]
            ).wait()

    return gather


def kernel(ids, embed_table, W1, b1, W2, b2):
    B, L = ids.shape
    V = embed_table.shape[0]
    O = W2.shape[1]
    y = _compute_vocab_outputs(embed_table, W1, b1, W2, b2)
    idx = ids.reshape(-1).astype(jnp.int32)
    out = _make_gather(V, O, B * L)(y, idx)
    return out.reshape(B, L, O)
